# baseline (speedup 1.0000x reference)
import functools

import jax
import jax.numpy as jnp
from jax import lax
from jax.experimental import pallas as pl
from jax.experimental.pallas import tpu as pltpu
from jax.experimental.pallas import tpu_sc as plsc

VOCAB = 8192
TOK = 8192


def _sc_gather(table, idx_flat):
    info = plsc.get_sparse_core_info()
    nc, ns = info.num_cores, info.num_subcores
    nw = nc * ns
    b_per_w = TOK // nw
    ch = 8
    n_chunks = b_per_w // ch

    mesh = plsc.VectorSubcoreMesh(core_axis_name="c", subcore_axis_name="s")

    @functools.partial(
        pl.kernel,
        mesh=mesh,
        out_type=jax.ShapeDtypeStruct((TOK, VOCAB), jnp.float32),
        scratch_types=[
            pltpu.VMEM((ch,), jnp.int32),
            pltpu.VMEM((ch, VOCAB), jnp.float32),
            pltpu.SemaphoreType.DMA,
        ],
    )
    def gather_k(table_hbm, idx_hbm, out_hbm, idx_v, rows_v, sem):
        wid = lax.axis_index("s") * nc + lax.axis_index("c")
        base = wid * b_per_w

        def body(j, carry):
            off = base + j * ch
            pltpu.sync_copy(idx_hbm.at[pl.ds(off, ch)], idx_v)
            pltpu.async_copy(table_hbm.at[idx_v], rows_v, sem).wait()
            pltpu.sync_copy(rows_v, out_hbm.at[pl.ds(off, ch)])
            return carry

        lax.fori_loop(0, n_chunks, body, 0, unroll=False)

    return gather_k(table, idx_flat)


_K = 16
_GRID = TOK // _K


def _loss_body(idx_ref, gt_ref, *refs):
    del idx_ref
    row_refs, out_ref = refs[:_K], refs[_K]
    i = pl.program_id(0)
    x = jnp.concatenate(
        [r[...].reshape(1, VOCAB) for r in row_refs], axis=0
    )
    m = jnp.max(x, axis=-1, keepdims=True)
    lse = jnp.log(jnp.sum(jnp.exp(x - m), axis=-1, keepdims=True)) + m
    gt = gt_ref[0, 0, :]
    cols = lax.broadcasted_iota(jnp.int32, (_K, VOCAB), 1)
    picked = jnp.sum(
        jnp.where(cols == gt[:, None], x, 0.0), axis=-1, keepdims=True
    )
    part = jnp.sum(lse - picked).reshape(1, 1)

    @pl.when(i == 0)
    def _init():
        out_ref[...] = jnp.zeros((1, 1), jnp.float32)

    out_ref[...] += part


def _row_spec(k):
    return pl.BlockSpec(
        (1, 1, VOCAB), lambda i, idx_ref, _k=k: (idx_ref[i * _K + _k], 0, 0)
    )


def _tc_loss(table, idx_flat, gt_flat):
    table3d = table.reshape(VOCAB, 1, VOCAB)
    gt3d = gt_flat.reshape(_GRID, 1, _K)
    grid_spec = pltpu.PrefetchScalarGridSpec(
        num_scalar_prefetch=1,
        grid=(_GRID,),
        in_specs=[pl.BlockSpec((1, 1, _K), lambda i, idx_ref: (i, 0, 0))]
        + [_row_spec(k) for k in range(_K)],
        out_specs=pl.BlockSpec((1, 1), lambda i, idx_ref: (0, 0)),
    )
    acc = pl.pallas_call(
        _loss_body,
        grid_spec=grid_spec,
        out_shape=jax.ShapeDtypeStruct((1, 1), jnp.float32),
    )(idx_flat, gt3d, *([table3d] * _K))
    return acc[0, 0] / TOK


def kernel(idx, gt, table):
    idx_flat = idx.reshape(-1)
    logits2d = _sc_gather(table, idx_flat)
    loss = _tc_loss(table, idx_flat, gt.reshape(-1))
    return logits2d.reshape(idx.shape[0], idx.shape[1], VOCAB), loss


# speedup vs baseline: 2.7906x; 2.7906x over previous
import functools

import jax
import jax.numpy as jnp
from jax import lax
from jax.experimental import pallas as pl
from jax.experimental.pallas import tpu as pltpu
from jax.experimental.pallas import tpu_sc as plsc

VOCAB = 8192
TOK = 8192

_CH = 2
_NBUF = 4


def _sc_gather(table, idx2d):
    info = plsc.get_sparse_core_info()
    nc, ns = info.num_cores, info.num_subcores
    nw = nc * ns
    b_per_w = TOK // nw
    n = b_per_w // _CH

    mesh = plsc.VectorSubcoreMesh(core_axis_name="c", subcore_axis_name="s")

    @functools.partial(
        pl.kernel,
        mesh=mesh,
        out_type=jax.ShapeDtypeStruct((TOK, VOCAB), jnp.float32),
        scratch_types=[
            pltpu.VMEM((n, _CH), jnp.int32),
            [pltpu.VMEM((_CH, VOCAB), jnp.float32) for _ in range(_NBUF)],
            [pltpu.SemaphoreType.DMA for _ in range(_NBUF)],
            [pltpu.SemaphoreType.DMA for _ in range(_NBUF)],
        ],
    )
    def gather_k(table_hbm, idx_hbm, out_hbm, idx_all, bufs, gsem, ssem):
        wid = lax.axis_index("s") * nc + lax.axis_index("c")
        base = wid * b_per_w

        pltpu.sync_copy(idx_hbm.at[pl.ds(wid * n, n)], idx_all)

        def g_start(j, b):
            pltpu.async_copy(table_hbm.at[idx_all.at[j]], bufs[b], gsem[b])

        def s_start(j, b):
            pltpu.async_copy(
                bufs[b], out_hbm.at[pl.ds(base + j * _CH, _CH)], ssem[b]
            )

        def s_wait(b):
            pltpu.make_async_copy(
                bufs[b], out_hbm.at[pl.ds(base, _CH)], ssem[b]
            ).wait()

        def g_wait(b):
            pltpu.make_async_copy(table_hbm.at[idx_all.at[0]], bufs[b],
                                  gsem[b]).wait()

        g_start(0, 0)
        g_start(1, 1)

        def body(j0, carry):
            for b in range(_NBUF):
                j = j0 * _NBUF + b
                g_wait(b)
                s_start(j, b)

                @pl.when(j >= 2)
                def _():
                    s_wait((b + 2) % _NBUF)

                @pl.when(j + 2 < n)
                def _():
                    g_start(j + 2, (b + 2) % _NBUF)

            return carry

        lax.fori_loop(0, n // _NBUF, body, 0, unroll=False)
        s_wait((n - 2) % _NBUF)
        s_wait((n - 1) % _NBUF)

    return gather_k(table, idx2d)


_ROWS = 256
_GRID = TOK // _ROWS


def _loss_body(gt_ref, x_ref, out_ref):
    i = pl.program_id(0)
    x = x_ref[...]
    m = jnp.max(x, axis=-1, keepdims=True)
    lse = jnp.log(jnp.sum(jnp.exp(x - m), axis=-1, keepdims=True)) + m
    gt = gt_ref[0, 0, :]
    cols = lax.broadcasted_iota(jnp.int32, (_ROWS, VOCAB), 1)
    picked = jnp.sum(
        jnp.where(cols == gt[:, None], x, 0.0), axis=-1, keepdims=True
    )
    part = jnp.sum(lse - picked).reshape(1, 1)

    @pl.when(i == 0)
    def _init():
        out_ref[...] = jnp.zeros((1, 1), jnp.float32)

    out_ref[...] += part


def _tc_loss(logits2d, gt_flat):
    gt3d = gt_flat.reshape(_GRID, 1, _ROWS)
    acc = pl.pallas_call(
        _loss_body,
        grid=(_GRID,),
        in_specs=[
            pl.BlockSpec((1, 1, _ROWS), lambda i: (i, 0, 0)),
            pl.BlockSpec((_ROWS, VOCAB), lambda i: (i, 0)),
        ],
        out_specs=pl.BlockSpec((1, 1), lambda i: (0, 0)),
        out_shape=jax.ShapeDtypeStruct((1, 1), jnp.float32),
    )(gt3d, logits2d)
    return acc[0, 0] / TOK


def kernel(idx, gt, table):
    idx2d = idx.reshape(-1, _CH)
    logits2d = _sc_gather(table, idx2d)
    loss = _tc_loss(logits2d, gt.reshape(-1))
    return logits2d.reshape(idx.shape[0], idx.shape[1], VOCAB), loss
